# baseline (speedup 1.0000x reference)
import functools

import jax
import jax.numpy as jnp
from jax import lax
from jax.experimental import pallas as pl
from jax.experimental.pallas import tpu as pltpu
from jax.experimental.pallas import tpu_sc as plsc

_VOCAB = 30000
_EMB = 128
_HID = 512
_NCLASS = 1000
_NCLASS_PAD = 1024
_B = 64
_S = 512

_NC = 2
_NS = 16
_NW = _NC * _NS
_TOKENS = _B * _S
_PER_W = _TOKENS // _NW
_CHUNK = 512

_CH = 16
_NCHUNK = _S // _CH


def _sc_gather(idx, table):
  mesh = plsc.VectorSubcoreMesh(core_axis_name="c", subcore_axis_name="s")

  @functools.partial(
      pl.kernel,
      mesh=mesh,
      out_type=jax.ShapeDtypeStruct((_TOKENS, _EMB), jnp.float32),
      scratch_types=[
          pltpu.VMEM((_CHUNK,), jnp.int32),
          pltpu.VMEM((_CHUNK, _EMB), jnp.float32),
          pltpu.SemaphoreType.DMA,
      ],
  )
  def gather_kernel(idx_hbm, table_hbm, out_hbm, idx_v, rows_v, sem):
    wid = lax.axis_index("s") * _NC + lax.axis_index("c")
    base = wid * _PER_W
    for ci in range(_PER_W // _CHUNK):
      off = base + ci * _CHUNK
      pltpu.sync_copy(idx_hbm.at[pl.ds(off, _CHUNK)], idx_v)
      pltpu.async_copy(table_hbm.at[idx_v], rows_v, sem).wait()
      pltpu.sync_copy(rows_v, out_hbm.at[pl.ds(off, _CHUNK)])

  return gather_kernel(idx, table)


def _sig(v):
  return 0.5 * jnp.tanh(0.5 * v) + 0.5


def _lstm_body(xe_ref, wx_ref, wh_ref, b_ref, fcw_ref, fcb_ref, out_ref,
               gxa_scr, gxb_scr, ha_scr, hb_scr, ca_scr, cb_scr):
  ha_scr[...] = jnp.zeros_like(ha_scr)
  hb_scr[...] = jnp.zeros_like(hb_scr)
  ca_scr[...] = jnp.zeros_like(ca_scr)
  cb_scr[...] = jnp.zeros_like(cb_scr)

  def fill_gx(gx, ci):
    xc = xe_ref[pl.ds(ci * _CH * _B, _CH * _B), :].astype(jnp.bfloat16)
    gx[...] = (
        jnp.dot(xc, wx_ref[...], preferred_element_type=jnp.float32)
        + b_ref[...]
    )

  fill_gx(gxa_scr, 0)

  def run_steps(cur, nxt, ci_next):
    def step_body(j, _):
      row = j * _B
      xn = xe_ref[pl.ds((ci_next * _CH + j) * _B, _B), :].astype(jnp.bfloat16)
      nxt[pl.ds(row, _B), :] = (
          jnp.dot(xn, wx_ref[...], preferred_element_type=jnp.float32)
          + b_ref[...]
      )

      for lo, h_r, c_r in ((0, ha_scr, ca_scr), (_B // 2, hb_scr, cb_scr)):
        hh = h_r[...].astype(jnp.bfloat16)
        c = c_r[...]

        def gate_chunk(k):
          return (
              cur[pl.ds(row + lo, _B // 2), k * _HID:(k + 1) * _HID]
              + jnp.dot(hh, wh_ref[:, k * _HID:(k + 1) * _HID],
                        preferred_element_type=jnp.float32)
          )

        i_g = _sig(gate_chunk(0))
        f_g = _sig(gate_chunk(1))
        g_g = jnp.tanh(gate_chunk(2))
        o_g = _sig(gate_chunk(3))
        c_new = f_g * c + i_g * g_g
        h_new = o_g * jnp.tanh(c_new)
        h_r[...] = h_new
        c_r[...] = c_new
      return 0

    lax.fori_loop(0, _CH, step_body, 0, unroll=4)

  def pair_body(m, _):
    ci1 = 2 * m + 1
    run_steps(gxa_scr, gxb_scr, ci1)
    run_steps(gxb_scr, gxa_scr, lax.rem(ci1 + 1, _NCHUNK))
    return 0

  lax.fori_loop(0, _NCHUNK // 2, pair_body, 0)

  out_ref[0:_B // 2, :] = (
      jnp.dot(ha_scr[...], fcw_ref[...], preferred_element_type=jnp.float32)
      + fcb_ref[...]
  )
  out_ref[_B // 2:_B, :] = (
      jnp.dot(hb_scr[...], fcw_ref[...], preferred_element_type=jnp.float32)
      + fcb_ref[...]
  )


def _lstm_fc(xe, wx, wh, b, fcw, fcb):
  return pl.pallas_call(
      _lstm_body,
      out_shape=jax.ShapeDtypeStruct((_B, _NCLASS_PAD), jnp.float32),
      scratch_shapes=[
          pltpu.VMEM((_CH * _B, 4 * _HID), jnp.float32),
          pltpu.VMEM((_CH * _B, 4 * _HID), jnp.float32),
          pltpu.VMEM((_B // 2, _HID), jnp.float32),
          pltpu.VMEM((_B // 2, _HID), jnp.float32),
          pltpu.VMEM((_B // 2, _HID), jnp.float32),
          pltpu.VMEM((_B // 2, _HID), jnp.float32),
      ],
  )(xe, wx, wh, b, fcw, fcb)


@jax.jit
def kernel(x, emb, W_ih, W_hh, b_ih, b_hh, fc_W, fc_b):
  idx = jnp.transpose(x, (1, 0)).reshape(_TOKENS)
  xe = _sc_gather(idx, emb)

  wx = W_ih.T.astype(jnp.bfloat16)
  wh = W_hh.T.astype(jnp.bfloat16)
  b = (b_ih + b_hh).reshape(1, 4 * _HID)
  fcw = jnp.pad(fc_W.T, ((0, 0), (0, _NCLASS_PAD - _NCLASS)))
  fcb = jnp.pad(fc_b, (0, _NCLASS_PAD - _NCLASS)).reshape(1, _NCLASS_PAD)

  out = _lstm_fc(xe, wx, wh, b, fcw, fcb)
  return out[:, :_NCLASS]


# speedup vs baseline: 1.5417x; 1.5417x over previous
import functools

import jax
import jax.numpy as jnp
from jax import lax
from jax.experimental import pallas as pl
from jax.experimental.pallas import tpu as pltpu
from jax.experimental.pallas import tpu_sc as plsc

_VOCAB = 30000
_EMB = 128
_HID = 512
_NCLASS = 1000
_NCLASS_PAD = 1024
_B = 64
_S = 512

_NC = 2
_NS = 16
_NW = _NC * _NS
_TOKENS = _B * _S
_PER_W = _TOKENS // _NW
_CHUNK = 512

_CH = 16
_NCHUNK = _S // _CH


def _sc_gather(idx, table):
  mesh = plsc.VectorSubcoreMesh(core_axis_name="c", subcore_axis_name="s")

  @functools.partial(
      pl.kernel,
      mesh=mesh,
      out_type=jax.ShapeDtypeStruct((_TOKENS, _EMB), jnp.float32),
      scratch_types=[
          pltpu.VMEM((_CHUNK,), jnp.int32),
          pltpu.VMEM((_CHUNK, _EMB), jnp.float32),
          pltpu.SemaphoreType.DMA,
      ],
  )
  def gather_kernel(idx_hbm, table_hbm, out_hbm, idx_v, rows_v, sem):
    wid = lax.axis_index("s") * _NC + lax.axis_index("c")
    base = wid * _PER_W
    for ci in range(_PER_W // _CHUNK):
      off = base + ci * _CHUNK
      pltpu.sync_copy(idx_hbm.at[pl.ds(off, _CHUNK)], idx_v)
      pltpu.async_copy(table_hbm.at[idx_v], rows_v, sem).wait()
      pltpu.sync_copy(rows_v, out_hbm.at[pl.ds(off, _CHUNK)])

  return gather_kernel(idx, table)


def _sig(v):
  return 0.5 * jnp.tanh(0.5 * v) + 0.5


def _lstm_body(xe_ref, wx_ref, wh_ref, b_ref, fcw_ref, fcb_ref, out_ref,
               gxa_scr, gxb_scr, h_scr, c_scr):
  h_scr[...] = jnp.zeros_like(h_scr)
  c_scr[...] = jnp.zeros_like(c_scr)

  def fill_gx(gx, ci):
    xc = xe_ref[pl.ds(ci * _CH * _B, _CH * _B), :].astype(jnp.bfloat16)
    gx[...] = (
        jnp.dot(xc, wx_ref[...], preferred_element_type=jnp.float32)
        + b_ref[...]
    )

  fill_gx(gxa_scr, 0)

  def run_steps(cur, nxt, ci_next):
    def step_body(j, _):
      row = j * _B
      xn = xe_ref[pl.ds((ci_next * _CH + j) * _B, _B), :].astype(jnp.bfloat16)
      nxt[pl.ds(row, _B), :] = (
          jnp.dot(xn, wx_ref[...], preferred_element_type=jnp.float32)
          + b_ref[...]
      )

      hb = h_scr[...].astype(jnp.bfloat16)
      c = c_scr[...]

      def gate_chunk(k):
        return (
            cur[pl.ds(row, _B), k * _HID:(k + 1) * _HID]
            + jnp.dot(hb, wh_ref[:, k * _HID:(k + 1) * _HID],
                      preferred_element_type=jnp.float32)
        )

      i_g = _sig(gate_chunk(0))
      f_g = _sig(gate_chunk(1))
      g_g = jnp.tanh(gate_chunk(2))
      o_g = _sig(gate_chunk(3))
      c_new = f_g * c + i_g * g_g
      h_new = o_g * jnp.tanh(c_new)
      h_scr[...] = h_new
      c_scr[...] = c_new
      return 0

    lax.fori_loop(0, _CH, step_body, 0, unroll=4)

  def pair_body(m, _):
    ci1 = 2 * m + 1
    run_steps(gxa_scr, gxb_scr, ci1)
    run_steps(gxb_scr, gxa_scr, lax.rem(ci1 + 1, _NCHUNK))
    return 0

  lax.fori_loop(0, _NCHUNK // 2, pair_body, 0)

  out_ref[...] = (
      jnp.dot(h_scr[...], fcw_ref[...], preferred_element_type=jnp.float32)
      + fcb_ref[...]
  )


def _lstm_fc(xe, wx, wh, b, fcw, fcb):
  return pl.pallas_call(
      _lstm_body,
      out_shape=jax.ShapeDtypeStruct((_B, _NCLASS_PAD), jnp.float32),
      scratch_shapes=[
          pltpu.VMEM((_CH * _B, 4 * _HID), jnp.float32),
          pltpu.VMEM((_CH * _B, 4 * _HID), jnp.float32),
          pltpu.VMEM((_B, _HID), jnp.float32),
          pltpu.VMEM((_B, _HID), jnp.float32),
      ],
  )(xe, wx, wh, b, fcw, fcb)


@jax.jit
def kernel(x, emb, W_ih, W_hh, b_ih, b_hh, fc_W, fc_b):
  idx = jnp.transpose(x, (1, 0)).reshape(_TOKENS)
  xe = _sc_gather(idx, emb)

  wx = W_ih.T.astype(jnp.bfloat16)
  wh = W_hh.T.astype(jnp.bfloat16)
  b = (b_ih + b_hh).reshape(1, 4 * _HID)
  fcw = jnp.pad(fc_W.T, ((0, 0), (0, _NCLASS_PAD - _NCLASS)))
  fcb = jnp.pad(fc_b, (0, _NCLASS_PAD - _NCLASS)).reshape(1, _NCLASS_PAD)

  out = _lstm_fc(xe, wx, wh, b, fcw, fcb)
  return out[:, :_NCLASS]
